# Initial kernel scaffold; baseline (speedup 1.0000x reference)
import jax
import jax.numpy as jnp
from jax.experimental import pallas as pl


def kernel(feats, edge_index, W1, b1, W2, b2):
    raise NotImplementedError("write your pallas kernel here")


# speedup vs baseline: 4.3363x; 4.3363x over previous
import functools

import jax
import jax.numpy as jnp
from jax import lax
from jax.experimental import pallas as pl
from jax.experimental.pallas import tpu as pltpu
from jax.experimental.pallas import tpu_sc as plsc

N_NODES = 10000
D_IN = 128
D_HIDDEN = 256
D_OUT = 64
SAMPLE = 4
ORDER = 8
P_DROP_NODE = 0.5

NC = 2
NS = 16
EDGE_BLOCK = 128
PAD_ROWS = 112
ACC_ROWS = N_NODES + PAD_ROWS
ROWS_PER_TILE = ACC_ROWS // NS
SAMPLES_PER_CORE = SAMPLE // NC

_MESH = plsc.VectorSubcoreMesh(core_axis_name="c", subcore_axis_name="s")


def _pad_edges(idx, n_edges_pad, pad_base):
    n = idx.shape[0]
    pad = jnp.arange(n_edges_pad - n, dtype=jnp.int32) % PAD_ROWS + pad_base
    return jnp.concatenate([idx, pad])


def _make_round_kernel(n_edges_pad, n_samples=SAMPLE):
    blocks_per_tile = n_edges_pad // EDGE_BLOCK // NS
    samples_per_core = n_samples // NC

    def body(h_hbm, src_hbm, dst_hbm, zeros_hbm, agg_hbm,
             acc, src_v, dst_v, rows_v, sem):
        c = lax.axis_index("c")
        t = lax.axis_index("s")
        row0 = t * ROWS_PER_TILE
        for si in range(samples_per_core):
            s = c * samples_per_core + si
            pltpu.sync_copy(zeros_hbm.at[pl.ds(row0, ROWS_PER_TILE)],
                            acc.at[pl.ds(row0, ROWS_PER_TILE)])
            plsc.subcore_barrier()

            def blk(b, carry):
                off = (t * blocks_per_tile + b) * EDGE_BLOCK
                pltpu.sync_copy(src_hbm.at[s, pl.ds(off, EDGE_BLOCK)], src_v)
                pltpu.sync_copy(dst_hbm.at[s, pl.ds(off, EDGE_BLOCK)], dst_v)
                pltpu.async_copy(h_hbm.at[src_v], rows_v, sem).wait()
                pltpu.sync_copy(rows_v, acc.at[dst_v], add=True)
                return carry

            lax.fori_loop(0, blocks_per_tile, blk, 0)
            plsc.subcore_barrier()
            pltpu.sync_copy(acc.at[pl.ds(row0, ROWS_PER_TILE)],
                            agg_hbm.at[s, pl.ds(row0, ROWS_PER_TILE)])

    return pl.kernel(
        body,
        out_type=jax.ShapeDtypeStruct((n_samples, ACC_ROWS, D_IN), jnp.float32),
        mesh=_MESH,
        scratch_types=[
            pltpu.VMEM_SHARED((ACC_ROWS, D_IN), jnp.float32),
            pltpu.VMEM((EDGE_BLOCK,), jnp.int32),
            pltpu.VMEM((EDGE_BLOCK,), jnp.int32),
            pltpu.VMEM((EDGE_BLOCK, D_IN), jnp.float32),
            pltpu.SemaphoreType.DMA,
        ],
    )


_RB = 2000


def _norm_body(deg_ref, ns_ref, nd_ref, c_ref):
    d = deg_ref[...]
    nd = lax.rsqrt(jnp.maximum(d[:, 0:1], 1.0))
    ns = lax.rsqrt(jnp.maximum(d[:, 1:2], 1.0))
    ns_ref[...] = ns
    nd_ref[...] = nd
    c_ref[...] = ns * nd


def _norm_call(deg2):
    grid = (N_NODES // _RB,)
    return pl.pallas_call(
        _norm_body,
        grid=grid,
        in_specs=[pl.BlockSpec((_RB, 2), lambda i: (i, 0))],
        out_specs=[pl.BlockSpec((_RB, 1), lambda i: (i, 0))] * 3,
        out_shape=[jax.ShapeDtypeStruct((N_NODES, 1), jnp.float32)] * 3,
    )(deg2)


def _init_body(feats_ref, mask_ref, ns_ref, h_ref, y_ref):
    x = feats_ref[...] * mask_ref[0]
    y_ref[...] = x[None]
    h_ref[...] = x * ns_ref[...]


def _init_call(feats, masks, ns):
    grid = (SAMPLE, N_NODES // _RB)
    return pl.pallas_call(
        _init_body,
        grid=grid,
        in_specs=[
            pl.BlockSpec((_RB, D_IN), lambda s, i: (i, 0)),
            pl.BlockSpec((1, _RB, 1), lambda s, i: (s, i, 0)),
            pl.BlockSpec((_RB, 1), lambda s, i: (i, 0)),
        ],
        out_specs=[
            pl.BlockSpec((_RB, D_IN), lambda s, i: (s * (N_NODES // _RB) + i, 0)),
            pl.BlockSpec((1, _RB, D_IN), lambda s, i: (s, i, 0)),
        ],
        out_shape=[
            jax.ShapeDtypeStruct((SAMPLE * N_NODES, D_IN), jnp.float32),
            jax.ShapeDtypeStruct((SAMPLE, N_NODES, D_IN), jnp.float32),
        ],
    )(feats, masks, ns)


def _scale_body(agg_ref, nd_ref, c_ref, y_ref, h_ref, ynew_ref):
    a = agg_ref[0]
    h_ref[...] = a * c_ref[...]
    ynew_ref[...] = y_ref[...] + a * nd_ref[...]


def _scale_call(agg, nd, cvec, y):
    grid = (SAMPLE, N_NODES // _RB)
    return pl.pallas_call(
        _scale_body,
        grid=grid,
        in_specs=[
            pl.BlockSpec((1, _RB, D_IN), lambda s, i: (s, i, 0)),
            pl.BlockSpec((_RB, 1), lambda s, i: (i, 0)),
            pl.BlockSpec((_RB, 1), lambda s, i: (i, 0)),
            pl.BlockSpec((1, _RB, D_IN), lambda s, i: (s, i, 0)),
        ],
        out_specs=[
            pl.BlockSpec((_RB, D_IN), lambda s, i: (s * (N_NODES // _RB) + i, 0)),
            pl.BlockSpec((1, _RB, D_IN), lambda s, i: (s, i, 0)),
        ],
        out_shape=[
            jax.ShapeDtypeStruct((SAMPLE * N_NODES, D_IN), jnp.float32),
            jax.ShapeDtypeStruct((SAMPLE, N_NODES, D_IN), jnp.float32),
        ],
    )(agg, nd, cvec, y)


_MB = 1000


def _mlp_body(y_ref, w1_ref, b1_ref, w2_ref, b2_ref, out_ref):
    y = y_ref[0] * (1.0 / (ORDER + 1))
    h = jnp.dot(y, w1_ref[...], preferred_element_type=jnp.float32)
    h = jnp.maximum(h + b1_ref[...], 0.0)
    z = jnp.dot(h, w2_ref[...], preferred_element_type=jnp.float32)
    z = z + b2_ref[...]
    m = jnp.max(z, axis=-1, keepdims=True)
    e = jnp.exp(z - m)
    lse = jnp.log(jnp.sum(e, axis=-1, keepdims=True))
    out_ref[0] = z - m - lse


def _mlp_call(y, W1, b1, W2, b2):
    grid = (SAMPLE, N_NODES // _MB)
    return pl.pallas_call(
        _mlp_body,
        grid=grid,
        in_specs=[
            pl.BlockSpec((1, _MB, D_IN), lambda s, i: (s, i, 0)),
            pl.BlockSpec((D_IN, D_HIDDEN), lambda s, i: (0, 0)),
            pl.BlockSpec((1, D_HIDDEN), lambda s, i: (0, 0)),
            pl.BlockSpec((D_HIDDEN, D_OUT), lambda s, i: (0, 0)),
            pl.BlockSpec((1, D_OUT), lambda s, i: (0, 0)),
        ],
        out_specs=pl.BlockSpec((1, _MB, D_OUT), lambda s, i: (s, i, 0)),
        out_shape=jax.ShapeDtypeStruct((SAMPLE, N_NODES, D_OUT), jnp.float32),
    )(y, W1, b1, W2, b2)


def kernel(feats, edge_index, W1, b1, W2, b2):
    src = edge_index[0].astype(jnp.int32)
    dst = edge_index[1].astype(jnp.int32)
    n_edges = src.shape[0]
    n_edges_pad = -(-n_edges // (EDGE_BLOCK * NS)) * (EDGE_BLOCK * NS)

    src_p = _pad_edges(src, n_edges_pad, 0)
    dst_p = _pad_edges(dst, n_edges_pad, N_NODES)
    src_pp = _pad_edges(src, n_edges_pad, N_NODES)

    offs = (jnp.arange(SAMPLE, dtype=jnp.int32) * N_NODES)[:, None]
    prop_src = src_p[None, :] + offs
    prop_dst = jnp.broadcast_to(dst_p, (SAMPLE, n_edges_pad))

    zeros_feat = jnp.zeros((ACC_ROWS, D_IN), jnp.float32)

    ones_h = jnp.ones((NC * N_NODES, D_IN), jnp.float32)
    deg_src = jnp.stack([src_p, src_p + N_NODES])
    deg_dst = jnp.stack([dst_p, src_pp])
    deg = _make_round_kernel(n_edges_pad, n_samples=NC)(
        ones_h, deg_src, deg_dst, zeros_feat)
    deg2 = jnp.stack([deg[0, :N_NODES, 0], deg[1, :N_NODES, 0]], axis=1)

    ns, nd, cvec = _norm_call(deg2)

    drop_key = jax.random.key(42)
    masks = jnp.stack([
        jax.random.bernoulli(jax.random.fold_in(drop_key, s),
                             1.0 - P_DROP_NODE, (N_NODES,))
        for s in range(SAMPLE)
    ]).astype(jnp.float32)[:, :, None]

    h, y = _init_call(feats, masks, ns)

    round_fn = _make_round_kernel(n_edges_pad)
    for _ in range(ORDER):
        agg = round_fn(h, prop_src, prop_dst, zeros_feat)
        h, y = _scale_call(agg, nd, cvec, y)

    return _mlp_call(y, W1, b1.reshape(1, -1), W2, b2.reshape(1, -1))


# speedup vs baseline: 7.7752x; 1.7930x over previous
import functools

import jax
import jax.numpy as jnp
from jax import lax
from jax.experimental import pallas as pl
from jax.experimental.pallas import tpu as pltpu
from jax.experimental.pallas import tpu_sc as plsc

N_NODES = 10000
D_IN = 128
D_HIDDEN = 256
D_OUT = 64
SAMPLE = 4
ORDER = 8
P_DROP_NODE = 0.5

NC = 2
NS = 16
EDGE_BLOCK = 128
PAD_ROWS = 112
ACC_ROWS = N_NODES + PAD_ROWS
ROWS_PER_TILE = ACC_ROWS // NS
SAMPLES_PER_CORE = SAMPLE // NC

_MESH = plsc.VectorSubcoreMesh(core_axis_name="c", subcore_axis_name="s")


def _pad_edges(idx, n_edges_pad, pad_base):
    n = idx.shape[0]
    pad = jnp.arange(n_edges_pad - n, dtype=jnp.int32) % PAD_ROWS + pad_base
    return jnp.concatenate([idx, pad])


NBUF = 4


def _make_round_kernel(n_edges_pad, n_samples=SAMPLE):
    blocks_per_tile = n_edges_pad // EDGE_BLOCK // NS
    samples_per_core = n_samples // NC
    n_groups = blocks_per_tile // NBUF
    assert blocks_per_tile % NBUF == 0

    n_quarters = 4
    qb = blocks_per_tile // n_quarters

    def body(h_hbm, src_hbm, dst_hbm, zeros_hbm, agg_hbm,
             acc, src_q, dst_q, rows, gsem, ssem):
        c = lax.axis_index("c")
        t = lax.axis_index("s")
        row0 = t * ROWS_PER_TILE

        def gstart(j):
            pltpu.async_copy(h_hbm.at[src_q.at[j]], rows.at[j % 2], gsem)

        def gwait():
            pltpu.make_async_copy(h_hbm.at[src_q.at[0]], rows.at[0], gsem).wait()

        def sstart(j):
            pltpu.async_copy(rows.at[j % 2], acc.at[dst_q.at[j]], ssem, add=True)

        def swait():
            pltpu.make_async_copy(rows.at[0], acc.at[dst_q.at[0]], ssem).wait()

        for si in range(samples_per_core):
            s = c * samples_per_core + si
            pltpu.sync_copy(zeros_hbm.at[pl.ds(row0, ROWS_PER_TILE)],
                            acc.at[pl.ds(row0, ROWS_PER_TILE)])
            plsc.subcore_barrier()
            for q in range(n_quarters):
                pltpu.sync_copy(src_hbm.at[s, t, pl.ds(q * qb, qb)], src_q)
                pltpu.sync_copy(dst_hbm.at[s, t, pl.ds(q * qb, qb)], dst_q)
                gstart(0)
                gwait()
                sstart(0)
                gstart(1)

                def blk(j, carry):
                    gwait()
                    sstart(j)
                    swait()
                    gstart(j + 1)
                    return carry

                lax.fori_loop(1, qb - 1, blk, 0)
                gwait()
                sstart(qb - 1)
                swait()
                swait()
            plsc.subcore_barrier()
            pltpu.sync_copy(acc.at[pl.ds(row0, ROWS_PER_TILE)],
                            agg_hbm.at[s, pl.ds(row0, ROWS_PER_TILE)])

    return pl.kernel(
        body,
        out_type=jax.ShapeDtypeStruct((n_samples, ACC_ROWS, D_IN), jnp.float32),
        mesh=_MESH,
        scratch_types=[
            pltpu.VMEM_SHARED((ACC_ROWS, D_IN), jnp.float32),
            pltpu.VMEM((blocks_per_tile // n_quarters, EDGE_BLOCK), jnp.int32),
            pltpu.VMEM((blocks_per_tile // n_quarters, EDGE_BLOCK), jnp.int32),
            pltpu.VMEM((2, EDGE_BLOCK, D_IN), jnp.float32),
            pltpu.SemaphoreType.DMA,
            pltpu.SemaphoreType.DMA,
        ],
    )


_RB = 2000


def _norm_body(deg_ref, ns_ref, nd_ref, c_ref):
    d = deg_ref[...]
    nd = lax.rsqrt(jnp.maximum(d[:, 0:1], 1.0))
    ns = lax.rsqrt(jnp.maximum(d[:, 1:2], 1.0))
    ns_ref[...] = ns
    nd_ref[...] = nd
    c_ref[...] = ns * nd


def _norm_call(deg2):
    grid = (N_NODES // _RB,)
    return pl.pallas_call(
        _norm_body,
        grid=grid,
        in_specs=[pl.BlockSpec((_RB, 2), lambda i: (i, 0))],
        out_specs=[pl.BlockSpec((_RB, 1), lambda i: (i, 0))] * 3,
        out_shape=[jax.ShapeDtypeStruct((N_NODES, 1), jnp.float32)] * 3,
    )(deg2)


def _init_body(feats_ref, mask_ref, ns_ref, h_ref, y_ref):
    x = feats_ref[...] * mask_ref[0]
    y_ref[...] = x[None]
    h_ref[...] = x * ns_ref[...]


def _init_call(feats, masks, ns):
    grid = (SAMPLE, N_NODES // _RB)
    return pl.pallas_call(
        _init_body,
        grid=grid,
        in_specs=[
            pl.BlockSpec((_RB, D_IN), lambda s, i: (i, 0)),
            pl.BlockSpec((1, _RB, 1), lambda s, i: (s, i, 0)),
            pl.BlockSpec((_RB, 1), lambda s, i: (i, 0)),
        ],
        out_specs=[
            pl.BlockSpec((_RB, D_IN), lambda s, i: (s * (N_NODES // _RB) + i, 0)),
            pl.BlockSpec((1, _RB, D_IN), lambda s, i: (s, i, 0)),
        ],
        out_shape=[
            jax.ShapeDtypeStruct((SAMPLE * N_NODES, D_IN), jnp.float32),
            jax.ShapeDtypeStruct((SAMPLE, N_NODES, D_IN), jnp.float32),
        ],
    )(feats, masks, ns)


def _scale_body(agg_ref, nd_ref, c_ref, y_ref, h_ref, ynew_ref):
    a = agg_ref[0]
    h_ref[...] = a * c_ref[...]
    ynew_ref[...] = y_ref[...] + a * nd_ref[...]


def _scale_call(agg, nd, cvec, y):
    grid = (SAMPLE, N_NODES // _RB)
    return pl.pallas_call(
        _scale_body,
        grid=grid,
        in_specs=[
            pl.BlockSpec((1, _RB, D_IN), lambda s, i: (s, i, 0)),
            pl.BlockSpec((_RB, 1), lambda s, i: (i, 0)),
            pl.BlockSpec((_RB, 1), lambda s, i: (i, 0)),
            pl.BlockSpec((1, _RB, D_IN), lambda s, i: (s, i, 0)),
        ],
        out_specs=[
            pl.BlockSpec((_RB, D_IN), lambda s, i: (s * (N_NODES // _RB) + i, 0)),
            pl.BlockSpec((1, _RB, D_IN), lambda s, i: (s, i, 0)),
        ],
        out_shape=[
            jax.ShapeDtypeStruct((SAMPLE * N_NODES, D_IN), jnp.float32),
            jax.ShapeDtypeStruct((SAMPLE, N_NODES, D_IN), jnp.float32),
        ],
    )(agg, nd, cvec, y)


_MB = 1000


def _mlp_body(y_ref, w1_ref, b1_ref, w2_ref, b2_ref, out_ref):
    y = y_ref[0] * (1.0 / (ORDER + 1))
    h = jnp.dot(y, w1_ref[...], preferred_element_type=jnp.float32)
    h = jnp.maximum(h + b1_ref[...], 0.0)
    z = jnp.dot(h, w2_ref[...], preferred_element_type=jnp.float32)
    z = z + b2_ref[...]
    m = jnp.max(z, axis=-1, keepdims=True)
    e = jnp.exp(z - m)
    lse = jnp.log(jnp.sum(e, axis=-1, keepdims=True))
    out_ref[0] = z - m - lse


def _mlp_call(y, W1, b1, W2, b2):
    grid = (SAMPLE, N_NODES // _MB)
    return pl.pallas_call(
        _mlp_body,
        grid=grid,
        in_specs=[
            pl.BlockSpec((1, _MB, D_IN), lambda s, i: (s, i, 0)),
            pl.BlockSpec((D_IN, D_HIDDEN), lambda s, i: (0, 0)),
            pl.BlockSpec((1, D_HIDDEN), lambda s, i: (0, 0)),
            pl.BlockSpec((D_HIDDEN, D_OUT), lambda s, i: (0, 0)),
            pl.BlockSpec((1, D_OUT), lambda s, i: (0, 0)),
        ],
        out_specs=pl.BlockSpec((1, _MB, D_OUT), lambda s, i: (s, i, 0)),
        out_shape=jax.ShapeDtypeStruct((SAMPLE, N_NODES, D_OUT), jnp.float32),
    )(y, W1, b1, W2, b2)


def kernel(feats, edge_index, W1, b1, W2, b2):
    src = edge_index[0].astype(jnp.int32)
    dst = edge_index[1].astype(jnp.int32)
    n_edges = src.shape[0]
    quantum = EDGE_BLOCK * NS * NBUF
    n_edges_pad = -(-n_edges // quantum) * quantum

    src_p = _pad_edges(src, n_edges_pad, 0)
    dst_p = _pad_edges(dst, n_edges_pad, N_NODES)
    src_pp = _pad_edges(src, n_edges_pad, N_NODES)

    blocks_per_tile = n_edges_pad // EDGE_BLOCK // NS
    idx_shape = (NS, blocks_per_tile, EDGE_BLOCK)
    offs = (jnp.arange(SAMPLE, dtype=jnp.int32) * N_NODES)[:, None, None, None]
    prop_src = src_p.reshape(idx_shape)[None] + offs
    prop_dst = jnp.broadcast_to(dst_p.reshape(idx_shape), (SAMPLE,) + idx_shape)

    zeros_feat = jnp.zeros((ACC_ROWS, D_IN), jnp.float32)

    ones_h = jnp.ones((NC * N_NODES, D_IN), jnp.float32)
    deg_src = jnp.stack([src_p, src_p + N_NODES]).reshape((NC,) + idx_shape)
    deg_dst = jnp.stack([dst_p, src_pp]).reshape((NC,) + idx_shape)
    deg = _make_round_kernel(n_edges_pad, n_samples=NC)(
        ones_h, deg_src, deg_dst, zeros_feat)
    deg2 = jnp.stack([deg[0, :N_NODES, 0], deg[1, :N_NODES, 0]], axis=1)

    ns, nd, cvec = _norm_call(deg2)

    drop_key = jax.random.key(42)
    masks = jnp.stack([
        jax.random.bernoulli(jax.random.fold_in(drop_key, s),
                             1.0 - P_DROP_NODE, (N_NODES,))
        for s in range(SAMPLE)
    ]).astype(jnp.float32)[:, :, None]

    h, y = _init_call(feats, masks, ns)

    round_fn = _make_round_kernel(n_edges_pad)
    for _ in range(ORDER):
        agg = round_fn(h, prop_src, prop_dst, zeros_feat)
        h, y = _scale_call(agg, nd, cvec, y)

    return _mlp_call(y, W1, b1.reshape(1, -1), W2, b2.reshape(1, -1))


# speedup vs baseline: 7.9863x; 1.0271x over previous
import functools

import jax
import jax.numpy as jnp
from jax import lax
from jax.experimental import pallas as pl
from jax.experimental.pallas import tpu as pltpu
from jax.experimental.pallas import tpu_sc as plsc

N_NODES = 10000
D_IN = 128
D_HIDDEN = 256
D_OUT = 64
SAMPLE = 4
ORDER = 8
P_DROP_NODE = 0.5

NC = 2
NS = 16
EDGE_BLOCK = 128
PAD_ROWS = 112
ACC_ROWS = N_NODES + PAD_ROWS
ROWS_PER_TILE = ACC_ROWS // NS
SAMPLES_PER_CORE = SAMPLE // NC

_MESH = plsc.VectorSubcoreMesh(core_axis_name="c", subcore_axis_name="s")


def _pad_edges(idx, n_edges_pad, pad_base):
    n = idx.shape[0]
    pad = jnp.arange(n_edges_pad - n, dtype=jnp.int32) % PAD_ROWS + pad_base
    return jnp.concatenate([idx, pad])


NBUF = 4


def _make_round_kernel(n_edges_pad, n_samples=SAMPLE):
    blocks_per_tile = n_edges_pad // EDGE_BLOCK // NS
    samples_per_core = n_samples // NC
    n_groups = blocks_per_tile // NBUF
    assert blocks_per_tile % NBUF == 0

    nblk = blocks_per_tile
    assert nblk >= 6

    def body(h_hbm, src_hbm, dst_hbm, zeros_hbm, agg_hbm,
             acc, srcr, dstr, rows, gsem, ssem, isem):
        c = lax.axis_index("c")
        t = lax.axis_index("s")
        row0 = t * ROWS_PER_TILE

        for si in range(samples_per_core):
            s = c * samples_per_core + si

            def istart(j):
                pltpu.async_copy(src_hbm.at[s, t, j], srcr.at[j % 2], isem)
                pltpu.async_copy(dst_hbm.at[s, t, j], dstr.at[j % 4], isem)

            def iwait():
                for _ in range(2):
                    pltpu.make_async_copy(src_hbm.at[s, t, 0], srcr.at[0],
                                          isem).wait()

            def gstart(j):
                pltpu.async_copy(h_hbm.at[srcr.at[j % 2]], rows.at[j % 3], gsem)

            def gwait():
                pltpu.make_async_copy(h_hbm.at[srcr.at[0]], rows.at[0],
                                      gsem).wait()

            def sstart(j):
                pltpu.async_copy(rows.at[j % 3], acc.at[dstr.at[j % 4]], ssem,
                                 add=True)

            def swait():
                pltpu.make_async_copy(rows.at[0], acc.at[dstr.at[0]],
                                      ssem).wait()

            istart(0)
            istart(1)
            pltpu.sync_copy(zeros_hbm.at[pl.ds(row0, ROWS_PER_TILE)],
                            acc.at[pl.ds(row0, ROWS_PER_TILE)])
            plsc.subcore_barrier()
            iwait()
            gstart(0)
            gwait(); sstart(0); iwait(); gstart(1); istart(2)
            gwait(); sstart(1); iwait(); gstart(2); istart(3)

            def blk(j, carry):
                gwait()
                sstart(j)
                swait()
                iwait()
                gstart(j + 1)
                istart(j + 2)
                return carry

            lax.fori_loop(2, nblk - 2, blk, 0)
            gwait(); sstart(nblk - 2); swait(); iwait(); gstart(nblk - 1)
            gwait(); sstart(nblk - 1); swait()
            swait()
            swait()
            plsc.subcore_barrier()
            pltpu.sync_copy(acc.at[pl.ds(row0, ROWS_PER_TILE)],
                            agg_hbm.at[s, pl.ds(row0, ROWS_PER_TILE)])

    return pl.kernel(
        body,
        out_type=jax.ShapeDtypeStruct((n_samples, ACC_ROWS, D_IN), jnp.float32),
        mesh=_MESH,
        scratch_types=[
            pltpu.VMEM_SHARED((ACC_ROWS, D_IN), jnp.float32),
            pltpu.VMEM((2, EDGE_BLOCK), jnp.int32),
            pltpu.VMEM((4, EDGE_BLOCK), jnp.int32),
            pltpu.VMEM((3, EDGE_BLOCK, D_IN), jnp.float32),
            pltpu.SemaphoreType.DMA,
            pltpu.SemaphoreType.DMA,
            pltpu.SemaphoreType.DMA,
        ],
    )


_RB = 2000


def _norm_body(deg_ref, ns_ref, nd_ref, c_ref):
    d = deg_ref[...]
    nd = lax.rsqrt(jnp.maximum(d[:, 0:1], 1.0))
    ns = lax.rsqrt(jnp.maximum(d[:, 1:2], 1.0))
    ns_ref[...] = ns
    nd_ref[...] = nd
    c_ref[...] = ns * nd


def _norm_call(deg2):
    grid = (N_NODES // _RB,)
    return pl.pallas_call(
        _norm_body,
        grid=grid,
        in_specs=[pl.BlockSpec((_RB, 2), lambda i: (i, 0))],
        out_specs=[pl.BlockSpec((_RB, 1), lambda i: (i, 0))] * 3,
        out_shape=[jax.ShapeDtypeStruct((N_NODES, 1), jnp.float32)] * 3,
    )(deg2)


def _init_body(feats_ref, mask_ref, ns_ref, h_ref, y_ref):
    x = feats_ref[...] * mask_ref[0]
    y_ref[...] = x[None]
    h_ref[...] = x * ns_ref[...]


def _init_call(feats, masks, ns):
    grid = (SAMPLE, N_NODES // _RB)
    return pl.pallas_call(
        _init_body,
        grid=grid,
        in_specs=[
            pl.BlockSpec((_RB, D_IN), lambda s, i: (i, 0)),
            pl.BlockSpec((1, _RB, 1), lambda s, i: (s, i, 0)),
            pl.BlockSpec((_RB, 1), lambda s, i: (i, 0)),
        ],
        out_specs=[
            pl.BlockSpec((_RB, D_IN), lambda s, i: (s * (N_NODES // _RB) + i, 0)),
            pl.BlockSpec((1, _RB, D_IN), lambda s, i: (s, i, 0)),
        ],
        out_shape=[
            jax.ShapeDtypeStruct((SAMPLE * N_NODES, D_IN), jnp.float32),
            jax.ShapeDtypeStruct((SAMPLE, N_NODES, D_IN), jnp.float32),
        ],
    )(feats, masks, ns)


def _scale_body(agg_ref, nd_ref, c_ref, y_ref, h_ref, ynew_ref):
    a = agg_ref[0]
    h_ref[...] = a * c_ref[...]
    ynew_ref[...] = y_ref[...] + a * nd_ref[...]


def _scale_call(agg, nd, cvec, y):
    grid = (SAMPLE, N_NODES // _RB)
    return pl.pallas_call(
        _scale_body,
        grid=grid,
        in_specs=[
            pl.BlockSpec((1, _RB, D_IN), lambda s, i: (s, i, 0)),
            pl.BlockSpec((_RB, 1), lambda s, i: (i, 0)),
            pl.BlockSpec((_RB, 1), lambda s, i: (i, 0)),
            pl.BlockSpec((1, _RB, D_IN), lambda s, i: (s, i, 0)),
        ],
        out_specs=[
            pl.BlockSpec((_RB, D_IN), lambda s, i: (s * (N_NODES // _RB) + i, 0)),
            pl.BlockSpec((1, _RB, D_IN), lambda s, i: (s, i, 0)),
        ],
        out_shape=[
            jax.ShapeDtypeStruct((SAMPLE * N_NODES, D_IN), jnp.float32),
            jax.ShapeDtypeStruct((SAMPLE, N_NODES, D_IN), jnp.float32),
        ],
    )(agg, nd, cvec, y)


_MB = 1000


def _mlp_body(y_ref, w1_ref, b1_ref, w2_ref, b2_ref, out_ref):
    y = y_ref[0] * (1.0 / (ORDER + 1))
    h = jnp.dot(y, w1_ref[...], preferred_element_type=jnp.float32)
    h = jnp.maximum(h + b1_ref[...], 0.0)
    z = jnp.dot(h, w2_ref[...], preferred_element_type=jnp.float32)
    z = z + b2_ref[...]
    m = jnp.max(z, axis=-1, keepdims=True)
    e = jnp.exp(z - m)
    lse = jnp.log(jnp.sum(e, axis=-1, keepdims=True))
    out_ref[0] = z - m - lse


def _mlp_call(y, W1, b1, W2, b2):
    grid = (SAMPLE, N_NODES // _MB)
    return pl.pallas_call(
        _mlp_body,
        grid=grid,
        in_specs=[
            pl.BlockSpec((1, _MB, D_IN), lambda s, i: (s, i, 0)),
            pl.BlockSpec((D_IN, D_HIDDEN), lambda s, i: (0, 0)),
            pl.BlockSpec((1, D_HIDDEN), lambda s, i: (0, 0)),
            pl.BlockSpec((D_HIDDEN, D_OUT), lambda s, i: (0, 0)),
            pl.BlockSpec((1, D_OUT), lambda s, i: (0, 0)),
        ],
        out_specs=pl.BlockSpec((1, _MB, D_OUT), lambda s, i: (s, i, 0)),
        out_shape=jax.ShapeDtypeStruct((SAMPLE, N_NODES, D_OUT), jnp.float32),
    )(y, W1, b1, W2, b2)


def kernel(feats, edge_index, W1, b1, W2, b2):
    src = edge_index[0].astype(jnp.int32)
    dst = edge_index[1].astype(jnp.int32)
    n_edges = src.shape[0]
    quantum = EDGE_BLOCK * NS * NBUF
    n_edges_pad = -(-n_edges // quantum) * quantum

    src_p = _pad_edges(src, n_edges_pad, 0)
    dst_p = _pad_edges(dst, n_edges_pad, N_NODES)
    src_pp = _pad_edges(src, n_edges_pad, N_NODES)

    blocks_per_tile = n_edges_pad // EDGE_BLOCK // NS
    idx_shape = (NS, blocks_per_tile, EDGE_BLOCK)
    offs = (jnp.arange(SAMPLE, dtype=jnp.int32) * N_NODES)[:, None, None, None]
    prop_src = src_p.reshape(idx_shape)[None] + offs
    prop_dst = jnp.broadcast_to(dst_p.reshape(idx_shape), (SAMPLE,) + idx_shape)

    zeros_feat = jnp.zeros((ACC_ROWS, D_IN), jnp.float32)

    ones_h = jnp.ones((NC * N_NODES, D_IN), jnp.float32)
    deg_src = jnp.stack([src_p, src_p + N_NODES]).reshape((NC,) + idx_shape)
    deg_dst = jnp.stack([dst_p, src_pp]).reshape((NC,) + idx_shape)
    deg = _make_round_kernel(n_edges_pad, n_samples=NC)(
        ones_h, deg_src, deg_dst, zeros_feat)
    deg2 = jnp.stack([deg[0, :N_NODES, 0], deg[1, :N_NODES, 0]], axis=1)

    ns, nd, cvec = _norm_call(deg2)

    drop_key = jax.random.key(42)
    masks = jnp.stack([
        jax.random.bernoulli(jax.random.fold_in(drop_key, s),
                             1.0 - P_DROP_NODE, (N_NODES,))
        for s in range(SAMPLE)
    ]).astype(jnp.float32)[:, :, None]

    h, y = _init_call(feats, masks, ns)

    round_fn = _make_round_kernel(n_edges_pad)
    for _ in range(ORDER):
        agg = round_fn(h, prop_src, prop_dst, zeros_feat)
        h, y = _scale_call(agg, nd, cvec, y)

    return _mlp_call(y, W1, b1.reshape(1, -1), W2, b2.reshape(1, -1))


# speedup vs baseline: 7.9887x; 1.0003x over previous
import functools

import jax
import jax.numpy as jnp
from jax import lax
from jax.experimental import pallas as pl
from jax.experimental.pallas import tpu as pltpu
from jax.experimental.pallas import tpu_sc as plsc

N_NODES = 10000
D_IN = 128
D_HIDDEN = 256
D_OUT = 64
SAMPLE = 4
ORDER = 8
P_DROP_NODE = 0.5

NC = 2
NS = 16
EDGE_BLOCK = 128
PAD_ROWS = 112
ACC_ROWS = N_NODES + PAD_ROWS
ROWS_PER_TILE = ACC_ROWS // NS
SAMPLES_PER_CORE = SAMPLE // NC

_MESH = plsc.VectorSubcoreMesh(core_axis_name="c", subcore_axis_name="s")


def _pad_edges(idx, n_edges_pad, pad_base):
    n = idx.shape[0]
    pad = jnp.arange(n_edges_pad - n, dtype=jnp.int32) % PAD_ROWS + pad_base
    return jnp.concatenate([idx, pad])


EDGE_QUANTUM = 8


def _make_round_kernel(n_edges_pad, n_samples=SAMPLE):
    blocks_per_tile = n_edges_pad // EDGE_BLOCK // NS
    samples_per_core = n_samples // NC
    nblk = blocks_per_tile
    assert nblk >= 6

    def body(h_hbm, src_hbm, dst_hbm, zeros_hbm, agg_hbm,
             acc, srcr, dstr, rows, gsem, ssem, isem):
        c = lax.axis_index("c")
        t = lax.axis_index("s")
        row0 = t * ROWS_PER_TILE

        for si in range(samples_per_core):
            s = c * samples_per_core + si

            def istart(j):
                pltpu.async_copy(src_hbm.at[s, t, j], srcr.at[j % 2], isem)
                pltpu.async_copy(dst_hbm.at[s, t, j], dstr.at[j % 4], isem)

            def iwait():
                for _ in range(2):
                    pltpu.make_async_copy(src_hbm.at[s, t, 0], srcr.at[0],
                                          isem).wait()

            def gstart(j):
                pltpu.async_copy(h_hbm.at[srcr.at[j % 2]], rows.at[j % 3], gsem)

            def gwait():
                pltpu.make_async_copy(h_hbm.at[srcr.at[0]], rows.at[0],
                                      gsem).wait()

            def sstart(j):
                pltpu.async_copy(rows.at[j % 3], acc.at[dstr.at[j % 4]], ssem,
                                 add=True)

            def swait():
                pltpu.make_async_copy(rows.at[0], acc.at[dstr.at[0]],
                                      ssem).wait()

            istart(0)
            istart(1)
            pltpu.sync_copy(zeros_hbm.at[pl.ds(row0, ROWS_PER_TILE)],
                            acc.at[pl.ds(row0, ROWS_PER_TILE)])
            plsc.subcore_barrier()
            iwait()
            gstart(0)
            gwait(); sstart(0); iwait(); gstart(1); istart(2)
            gwait(); sstart(1); iwait(); gstart(2); istart(3)

            def blk(j, carry):
                gwait()
                sstart(j)
                swait()
                iwait()
                gstart(j + 1)
                istart(j + 2)
                return carry

            lax.fori_loop(2, nblk - 2, blk, 0)
            gwait(); sstart(nblk - 2); swait(); iwait(); gstart(nblk - 1)
            gwait(); sstart(nblk - 1); swait()
            swait()
            swait()
            plsc.subcore_barrier()
            pltpu.sync_copy(acc.at[pl.ds(row0, ROWS_PER_TILE)],
                            agg_hbm.at[s, pl.ds(row0, ROWS_PER_TILE)])

    return pl.kernel(
        body,
        out_type=jax.ShapeDtypeStruct((n_samples, ACC_ROWS, D_IN), jnp.float32),
        mesh=_MESH,
        scratch_types=[
            pltpu.VMEM_SHARED((ACC_ROWS, D_IN), jnp.float32),
            pltpu.VMEM((2, EDGE_BLOCK), jnp.int32),
            pltpu.VMEM((4, EDGE_BLOCK), jnp.int32),
            pltpu.VMEM((3, EDGE_BLOCK, D_IN), jnp.float32),
            pltpu.SemaphoreType.DMA,
            pltpu.SemaphoreType.DMA,
            pltpu.SemaphoreType.DMA,
        ],
    )


_RB = 2000


def _norm_body(deg_ref, ns_ref, nd_ref, c_ref):
    d = deg_ref[...]
    nd = lax.rsqrt(jnp.maximum(d[:, 0:1], 1.0))
    ns = lax.rsqrt(jnp.maximum(d[:, 1:2], 1.0))
    ns_ref[...] = ns
    nd_ref[...] = nd
    c_ref[...] = ns * nd


def _norm_call(deg2):
    grid = (N_NODES // _RB,)
    return pl.pallas_call(
        _norm_body,
        grid=grid,
        in_specs=[pl.BlockSpec((_RB, 2), lambda i: (i, 0))],
        out_specs=[pl.BlockSpec((_RB, 1), lambda i: (i, 0))] * 3,
        out_shape=[jax.ShapeDtypeStruct((N_NODES, 1), jnp.float32)] * 3,
    )(deg2)


def _init_body(feats_ref, mask_ref, ns_ref, h_ref, y_ref):
    x = feats_ref[...] * mask_ref[0]
    y_ref[...] = x[None]
    h_ref[...] = x * ns_ref[...]


def _init_call(feats, masks, ns):
    grid = (SAMPLE, N_NODES // _RB)
    return pl.pallas_call(
        _init_body,
        grid=grid,
        in_specs=[
            pl.BlockSpec((_RB, D_IN), lambda s, i: (i, 0)),
            pl.BlockSpec((1, _RB, 1), lambda s, i: (s, i, 0)),
            pl.BlockSpec((_RB, 1), lambda s, i: (i, 0)),
        ],
        out_specs=[
            pl.BlockSpec((_RB, D_IN), lambda s, i: (s * (N_NODES // _RB) + i, 0)),
            pl.BlockSpec((1, _RB, D_IN), lambda s, i: (s, i, 0)),
        ],
        out_shape=[
            jax.ShapeDtypeStruct((SAMPLE * N_NODES, D_IN), jnp.float32),
            jax.ShapeDtypeStruct((SAMPLE, N_NODES, D_IN), jnp.float32),
        ],
    )(feats, masks, ns)


def _scale_body(agg_ref, nd_ref, c_ref, y_ref, h_ref, ynew_ref):
    a = agg_ref[0]
    h_ref[...] = a * c_ref[...]
    ynew_ref[...] = y_ref[...] + a * nd_ref[...]


def _scale_call(agg, nd, cvec, y):
    grid = (SAMPLE, N_NODES // _RB)
    return pl.pallas_call(
        _scale_body,
        grid=grid,
        in_specs=[
            pl.BlockSpec((1, _RB, D_IN), lambda s, i: (s, i, 0)),
            pl.BlockSpec((_RB, 1), lambda s, i: (i, 0)),
            pl.BlockSpec((_RB, 1), lambda s, i: (i, 0)),
            pl.BlockSpec((1, _RB, D_IN), lambda s, i: (s, i, 0)),
        ],
        out_specs=[
            pl.BlockSpec((_RB, D_IN), lambda s, i: (s * (N_NODES // _RB) + i, 0)),
            pl.BlockSpec((1, _RB, D_IN), lambda s, i: (s, i, 0)),
        ],
        out_shape=[
            jax.ShapeDtypeStruct((SAMPLE * N_NODES, D_IN), jnp.float32),
            jax.ShapeDtypeStruct((SAMPLE, N_NODES, D_IN), jnp.float32),
        ],
    )(agg, nd, cvec, y)


_MB = 1000


def _mlp_body(y_ref, w1_ref, b1_ref, w2_ref, b2_ref, out_ref):
    y = y_ref[0] * (1.0 / (ORDER + 1))
    h = jnp.dot(y, w1_ref[...], preferred_element_type=jnp.float32)
    h = jnp.maximum(h + b1_ref[...], 0.0)
    z = jnp.dot(h, w2_ref[...], preferred_element_type=jnp.float32)
    z = z + b2_ref[...]
    m = jnp.max(z, axis=-1, keepdims=True)
    e = jnp.exp(z - m)
    lse = jnp.log(jnp.sum(e, axis=-1, keepdims=True))
    out_ref[0] = z - m - lse


def _mlp_call(y, W1, b1, W2, b2):
    grid = (SAMPLE, N_NODES // _MB)
    return pl.pallas_call(
        _mlp_body,
        grid=grid,
        in_specs=[
            pl.BlockSpec((1, _MB, D_IN), lambda s, i: (s, i, 0)),
            pl.BlockSpec((D_IN, D_HIDDEN), lambda s, i: (0, 0)),
            pl.BlockSpec((1, D_HIDDEN), lambda s, i: (0, 0)),
            pl.BlockSpec((D_HIDDEN, D_OUT), lambda s, i: (0, 0)),
            pl.BlockSpec((1, D_OUT), lambda s, i: (0, 0)),
        ],
        out_specs=pl.BlockSpec((1, _MB, D_OUT), lambda s, i: (s, i, 0)),
        out_shape=jax.ShapeDtypeStruct((SAMPLE, N_NODES, D_OUT), jnp.float32),
    )(y, W1, b1, W2, b2)


def kernel(feats, edge_index, W1, b1, W2, b2):
    src = edge_index[0].astype(jnp.int32)
    dst = edge_index[1].astype(jnp.int32)
    n_edges = src.shape[0]
    quantum = EDGE_BLOCK * NS * EDGE_QUANTUM
    n_edges_pad = -(-n_edges // quantum) * quantum

    src_p = _pad_edges(src, n_edges_pad, 0)
    dst_p = _pad_edges(dst, n_edges_pad, N_NODES)
    src_pp = _pad_edges(src, n_edges_pad, N_NODES)

    blocks_per_tile = n_edges_pad // EDGE_BLOCK // NS
    idx_shape = (NS, blocks_per_tile, EDGE_BLOCK)
    offs = (jnp.arange(SAMPLE, dtype=jnp.int32) * N_NODES)[:, None, None, None]
    prop_src = src_p.reshape(idx_shape)[None] + offs
    prop_dst = jnp.broadcast_to(dst_p.reshape(idx_shape), (SAMPLE,) + idx_shape)

    zeros_feat = jnp.zeros((ACC_ROWS, D_IN), jnp.float32)

    ones_h = jnp.ones((NC * N_NODES, D_IN), jnp.float32)
    deg_src = jnp.stack([src_p, src_p + N_NODES]).reshape((NC,) + idx_shape)
    deg_dst = jnp.stack([dst_p, src_pp]).reshape((NC,) + idx_shape)
    deg = _make_round_kernel(n_edges_pad, n_samples=NC)(
        ones_h, deg_src, deg_dst, zeros_feat)
    deg2 = jnp.stack([deg[0, :N_NODES, 0], deg[1, :N_NODES, 0]], axis=1)

    ns, nd, cvec = _norm_call(deg2)

    drop_key = jax.random.key(42)
    masks = jnp.stack([
        jax.random.bernoulli(jax.random.fold_in(drop_key, s),
                             1.0 - P_DROP_NODE, (N_NODES,))
        for s in range(SAMPLE)
    ]).astype(jnp.float32)[:, :, None]

    h, y = _init_call(feats, masks, ns)

    round_fn = _make_round_kernel(n_edges_pad)
    for _ in range(ORDER):
        agg = round_fn(h, prop_src, prop_dst, zeros_feat)
        h, y = _scale_call(agg, nd, cvec, y)

    return _mlp_call(y, W1, b1.reshape(1, -1), W2, b2.reshape(1, -1))
